# Initial kernel scaffold; baseline (speedup 1.0000x reference)
import jax
import jax.numpy as jnp
from jax import lax
from jax.experimental import pallas as pl
from jax.experimental.pallas import tpu as pltpu
from jax.experimental.pallas import tpu_sc as plsc

N = 100000
D = 256
B = 512
NC = 2
NS = 16
CW = D // NC
CHUNK = 128
FULL = N // CHUNK
REM = N - FULL * CHUNK
K = -(-FULL // NS)
SEG_PER_SUB = B // NS


def _sc_body(data_ref, ids_ref, out_ref,
             acc_sh, cnt_sh,
             idx_v, rows_v, ones_v, zero_v, zcnt_v, acc_v, cnt_v):
    core = lax.axis_index("c")
    sid = lax.axis_index("s")
    col = core * CW

    zero16 = jnp.zeros((16,), jnp.float32)
    one16 = jnp.ones((16,), jnp.float32)

    for i in range(SEG_PER_SUB):
        for j in range(CW // 16):
            zero_v[i, pl.ds(j * 16, 16)] = zero16
        zcnt_v[i, pl.ds(0, 16)] = zero16
    for i in range(CHUNK):
        ones_v[i, pl.ds(0, 16)] = one16

    pltpu.sync_copy(zero_v, acc_sh.at[pl.ds(sid * SEG_PER_SUB, SEG_PER_SUB)])
    pltpu.sync_copy(zcnt_v, cnt_sh.at[pl.ds(sid * SEG_PER_SUB, SEG_PER_SUB)])
    plsc.subcore_barrier()

    def body(k, carry):
        c_idx = k * NS + sid

        @pl.when(c_idx < FULL)
        def _():
            off = c_idx * CHUNK
            pltpu.sync_copy(ids_ref.at[pl.ds(off, CHUNK)], idx_v)
            pltpu.sync_copy(data_ref.at[pl.ds(off, CHUNK), pl.ds(col, CW)],
                            rows_v)
            pltpu.sync_copy(rows_v, acc_sh.at[idx_v], add=True)
            pltpu.sync_copy(ones_v, cnt_sh.at[idx_v], add=True)

        return carry

    lax.fori_loop(0, K, body, 0)

    @pl.when(sid == 0)
    def _():
        pad16 = jnp.full((16,), B, jnp.int32)
        for j in range(REM // 16, CHUNK // 16):
            idx_v[pl.ds(j * 16, 16)] = pad16
        pltpu.sync_copy(ids_ref.at[pl.ds(FULL * CHUNK, REM)],
                        idx_v.at[pl.ds(0, REM)])
        pltpu.sync_copy(data_ref.at[pl.ds(FULL * CHUNK, REM), pl.ds(col, CW)],
                        rows_v.at[pl.ds(0, REM)])
        pltpu.sync_copy(rows_v, acc_sh.at[idx_v], add=True)
        pltpu.sync_copy(ones_v, cnt_sh.at[idx_v], add=True)

    plsc.subcore_barrier()

    seg0 = sid * SEG_PER_SUB
    pltpu.sync_copy(acc_sh.at[pl.ds(seg0, SEG_PER_SUB)], acc_v)
    pltpu.sync_copy(cnt_sh.at[pl.ds(seg0, SEG_PER_SUB)], cnt_v)
    for s in range(SEG_PER_SUB):
        c16 = cnt_v[s, pl.ds(0, 16)]
        inv = 1.0 / jnp.maximum(c16, 1.0)
        for j in range(CW // 16):
            acc_v[s, pl.ds(j * 16, 16)] = acc_v[s, pl.ds(j * 16, 16)] * inv
    pltpu.sync_copy(acc_v, out_ref.at[pl.ds(seg0, SEG_PER_SUB),
                                      pl.ds(col, CW)])


@jax.jit
def _segment_mean(data, ids32):
    mesh = plsc.VectorSubcoreMesh(core_axis_name="c", subcore_axis_name="s")
    return pl.kernel(
        _sc_body,
        out_type=jax.ShapeDtypeStruct((B, D), jnp.float32),
        mesh=mesh,
        scratch_types=[
            pltpu.VMEM_SHARED((B + 1, CW), jnp.float32),
            pltpu.VMEM_SHARED((B + 1, 16), jnp.float32),
            pltpu.VMEM((CHUNK,), jnp.int32),
            pltpu.VMEM((CHUNK, CW), jnp.float32),
            pltpu.VMEM((CHUNK, 16), jnp.float32),
            pltpu.VMEM((SEG_PER_SUB, CW), jnp.float32),
            pltpu.VMEM((SEG_PER_SUB, 16), jnp.float32),
            pltpu.VMEM((SEG_PER_SUB, CW), jnp.float32),
            pltpu.VMEM((SEG_PER_SUB, 16), jnp.float32),
        ],
    )(data, ids32)


def kernel(data, segment_ids, num_segments):
    del num_segments
    return _segment_mean(data, segment_ids.astype(jnp.int32))


# speedup vs baseline: 3.4105x; 3.4105x over previous
import jax
import jax.numpy as jnp
from jax import lax
from jax.experimental import pallas as pl
from jax.experimental.pallas import tpu as pltpu
from jax.experimental.pallas import tpu_sc as plsc

N = 100000
D = 256
B = 512
NC = 2
NS = 16
CW = D // NC
CHUNK = 128
FULL = N // CHUNK
REM = N - FULL * CHUNK
K = -(-FULL // NS)
SEG_PER_SUB = B // NS


def _sc_body(data_ref, ids_ref, out_ref,
             acc_sh, cnt_sh,
             idx_v, rows_v, ones_v, zero_v, acc_v, cnt_v):
    core = lax.axis_index("c")
    sid = lax.axis_index("s")
    col = core * CW

    zero16 = jnp.zeros((16,), jnp.float32)
    one16 = jnp.ones((16,), jnp.float32)

    for i in range(SEG_PER_SUB):
        for j in range(CW // 16):
            zero_v[i, pl.ds(j * 16, 16)] = zero16
    for i in range(CHUNK):
        for j in range(CW // 16):
            ones_v[i, pl.ds(j * 16, 16)] = one16

    pltpu.sync_copy(zero_v, acc_sh.at[pl.ds(sid * SEG_PER_SUB, SEG_PER_SUB)])
    pltpu.sync_copy(zero_v, cnt_sh.at[pl.ds(sid * SEG_PER_SUB, SEG_PER_SUB)])
    plsc.subcore_barrier()

    def body(k, carry):
        c_idx = k * NS + sid

        @pl.when(c_idx < FULL)
        def _():
            off = c_idx * CHUNK
            pltpu.sync_copy(ids_ref.at[pl.ds(off, CHUNK)], idx_v)
            pltpu.sync_copy(data_ref.at[pl.ds(off, CHUNK), pl.ds(col, CW)],
                            rows_v)
            pltpu.sync_copy(rows_v, acc_sh.at[idx_v], add=True)
            pltpu.sync_copy(ones_v, cnt_sh.at[idx_v], add=True)

        return carry

    lax.fori_loop(0, K, body, 0)

    @pl.when(sid == 0)
    def _():
        pad16 = jnp.full((16,), B, jnp.int32)
        for j in range(REM // 16, CHUNK // 16):
            idx_v[pl.ds(j * 16, 16)] = pad16
        pltpu.sync_copy(ids_ref.at[pl.ds(FULL * CHUNK, REM)],
                        idx_v.at[pl.ds(0, REM)])
        pltpu.sync_copy(data_ref.at[pl.ds(FULL * CHUNK, REM), pl.ds(col, CW)],
                        rows_v.at[pl.ds(0, REM)])
        pltpu.sync_copy(rows_v, acc_sh.at[idx_v], add=True)
        pltpu.sync_copy(ones_v, cnt_sh.at[idx_v], add=True)

    plsc.subcore_barrier()

    seg0 = sid * SEG_PER_SUB
    pltpu.sync_copy(acc_sh.at[pl.ds(seg0, SEG_PER_SUB)], acc_v)
    pltpu.sync_copy(cnt_sh.at[pl.ds(seg0, SEG_PER_SUB)], cnt_v)
    for s in range(SEG_PER_SUB):
        c16 = cnt_v[s, pl.ds(0, 16)]
        inv = 1.0 / jnp.maximum(c16, 1.0)
        for j in range(CW // 16):
            acc_v[s, pl.ds(j * 16, 16)] = acc_v[s, pl.ds(j * 16, 16)] * inv
    pltpu.sync_copy(acc_v, out_ref.at[pl.ds(seg0, SEG_PER_SUB),
                                      pl.ds(col, CW)])


@jax.jit
def _segment_mean(data, ids32):
    mesh = plsc.VectorSubcoreMesh(core_axis_name="c", subcore_axis_name="s")
    return pl.kernel(
        _sc_body,
        out_type=jax.ShapeDtypeStruct((B, D), jnp.float32),
        mesh=mesh,
        scratch_types=[
            pltpu.VMEM_SHARED((B + 1, CW), jnp.float32),
            pltpu.VMEM_SHARED((B + 1, CW), jnp.float32),
            pltpu.VMEM((CHUNK,), jnp.int32),
            pltpu.VMEM((CHUNK, CW), jnp.float32),
            pltpu.VMEM((CHUNK, CW), jnp.float32),
            pltpu.VMEM((SEG_PER_SUB, CW), jnp.float32),
            pltpu.VMEM((SEG_PER_SUB, CW), jnp.float32),
            pltpu.VMEM((SEG_PER_SUB, CW), jnp.float32),
        ],
    )(data, ids32)


def kernel(data, segment_ids, num_segments):
    del num_segments
    return _segment_mean(data, segment_ids.astype(jnp.int32))


# speedup vs baseline: 4.8426x; 1.4199x over previous
import jax
import jax.numpy as jnp
from jax import lax
from jax.experimental import pallas as pl
from jax.experimental.pallas import tpu as pltpu
from jax.experimental.pallas import tpu_sc as plsc

N = 100000
D = 256
B = 512
NC = 2
NS = 16
CW = D // NC
CHUNK = 128
FULL = N // CHUNK
REM = N - FULL * CHUNK
K = -(-FULL // NS)
KP = -(-K // 2)
SEG_PER_SUB = B // NS


def _sc_body(data_ref, ids_ref, out_ref,
             acc_sh, cnt_sh,
             idx0, idx1, rows0, rows1, ones_v, zero_v, acc_v, cnt_v,
             isem0, isem1, rsem0, rsem1):
    core = lax.axis_index("c")
    sid = lax.axis_index("s")
    col = core * CW

    zero16 = jnp.zeros((16,), jnp.float32)
    one16 = jnp.ones((16,), jnp.float32)

    for i in range(SEG_PER_SUB):
        for j in range(CW // 16):
            zero_v[i, pl.ds(j * 16, 16)] = zero16
    for i in range(CHUNK):
        for j in range(CW // 16):
            ones_v[i, pl.ds(j * 16, 16)] = one16

    pltpu.sync_copy(zero_v, acc_sh.at[pl.ds(sid * SEG_PER_SUB, SEG_PER_SUB)])
    pltpu.sync_copy(zero_v, cnt_sh.at[pl.ds(sid * SEG_PER_SUB, SEG_PER_SUB)])
    plsc.subcore_barrier()

    def start_gather(c_idx, ibuf, rbuf, isem, rsem):
        off = c_idx * CHUNK
        pltpu.async_copy(ids_ref.at[pl.ds(off, CHUNK)], ibuf, isem)
        pltpu.async_copy(data_ref.at[pl.ds(off, CHUNK), pl.ds(col, CW)],
                         rbuf, rsem)

    def wait_gather(ibuf, rbuf, isem, rsem):
        pltpu.make_async_copy(ids_ref.at[pl.ds(0, CHUNK)], ibuf, isem).wait()
        pltpu.make_async_copy(data_ref.at[pl.ds(0, CHUNK), pl.ds(0, CW)],
                              rbuf, rsem).wait()

    def scatter(ibuf, rbuf):
        pltpu.sync_copy(rbuf, acc_sh.at[ibuf], add=True)
        pltpu.sync_copy(ones_v, cnt_sh.at[ibuf], add=True)

    start_gather(sid, idx0, rows0, isem0, rsem0)

    def body(k, carry):
        c0 = (2 * k) * NS + sid
        c1 = c0 + NS
        c2 = c1 + NS

        @pl.when(c1 < FULL)
        def _():
            start_gather(c1, idx1, rows1, isem1, rsem1)

        @pl.when(c0 < FULL)
        def _():
            wait_gather(idx0, rows0, isem0, rsem0)
            scatter(idx0, rows0)

        @pl.when(c2 < FULL)
        def _():
            start_gather(c2, idx0, rows0, isem0, rsem0)

        @pl.when(c1 < FULL)
        def _():
            wait_gather(idx1, rows1, isem1, rsem1)
            scatter(idx1, rows1)

        return carry

    lax.fori_loop(0, KP, body, 0)

    @pl.when(sid == 0)
    def _():
        pad16 = jnp.full((16,), B, jnp.int32)
        for j in range(REM // 16, CHUNK // 16):
            idx0[pl.ds(j * 16, 16)] = pad16
        pltpu.sync_copy(ids_ref.at[pl.ds(FULL * CHUNK, REM)],
                        idx0.at[pl.ds(0, REM)])
        pltpu.sync_copy(data_ref.at[pl.ds(FULL * CHUNK, REM), pl.ds(col, CW)],
                        rows0.at[pl.ds(0, REM)])
        scatter(idx0, rows0)

    plsc.subcore_barrier()

    seg0 = sid * SEG_PER_SUB
    pltpu.sync_copy(acc_sh.at[pl.ds(seg0, SEG_PER_SUB)], acc_v)
    pltpu.sync_copy(cnt_sh.at[pl.ds(seg0, SEG_PER_SUB)], cnt_v)
    for s in range(SEG_PER_SUB):
        c16 = cnt_v[s, pl.ds(0, 16)]
        inv = 1.0 / jnp.maximum(c16, 1.0)
        for j in range(CW // 16):
            acc_v[s, pl.ds(j * 16, 16)] = acc_v[s, pl.ds(j * 16, 16)] * inv
    pltpu.sync_copy(acc_v, out_ref.at[pl.ds(seg0, SEG_PER_SUB),
                                      pl.ds(col, CW)])


@jax.jit
def _segment_mean(data, ids32):
    mesh = plsc.VectorSubcoreMesh(core_axis_name="c", subcore_axis_name="s")
    return pl.kernel(
        _sc_body,
        out_type=jax.ShapeDtypeStruct((B, D), jnp.float32),
        mesh=mesh,
        scratch_types=[
            pltpu.VMEM_SHARED((B + 1, CW), jnp.float32),
            pltpu.VMEM_SHARED((B + 1, CW), jnp.float32),
            pltpu.VMEM((CHUNK,), jnp.int32),
            pltpu.VMEM((CHUNK,), jnp.int32),
            pltpu.VMEM((CHUNK, CW), jnp.float32),
            pltpu.VMEM((CHUNK, CW), jnp.float32),
            pltpu.VMEM((CHUNK, CW), jnp.float32),
            pltpu.VMEM((SEG_PER_SUB, CW), jnp.float32),
            pltpu.VMEM((SEG_PER_SUB, CW), jnp.float32),
            pltpu.VMEM((SEG_PER_SUB, CW), jnp.float32),
            pltpu.SemaphoreType.DMA,
            pltpu.SemaphoreType.DMA,
            pltpu.SemaphoreType.DMA,
            pltpu.SemaphoreType.DMA,
        ],
    )(data, ids32)


def kernel(data, segment_ids, num_segments):
    del num_segments
    return _segment_mean(data, segment_ids.astype(jnp.int32))
